# Initial kernel scaffold; baseline (speedup 1.0000x reference)
import jax
import jax.numpy as jnp
from jax.experimental import pallas as pl


def kernel(x, emb, W2, b2, W31, b31, W32, b32):
    raise NotImplementedError("write your pallas kernel here")


# speedup vs baseline: 1.2696x; 1.2696x over previous
import functools

import jax
import jax.numpy as jnp
from jax import lax
from jax.experimental import pallas as pl
from jax.experimental.pallas import tpu as pltpu
from jax.experimental.pallas import tpu_sc as plsc

VOCAB = 10
VPAD = 16
NC = 2
NS = 16
NW = NC * NS
CHUNK = 128


def _table_body(emb_ref, w2_ref, b2_ref, w31_ref, b31_ref, w32_ref, b32_ref,
                out_ref):
    e = emb_ref[...]
    t = e * jax.nn.sigmoid(e)
    h = jnp.dot(t, w2_ref[...], preferred_element_type=jnp.float32) + b2_ref[...]
    h = h * jax.nn.sigmoid(h)
    lat = w31_ref.shape[1]
    out_ref[:, :lat] = (
        jnp.dot(h, w31_ref[...], preferred_element_type=jnp.float32) + b31_ref[...]
    )
    out_ref[:, lat:] = (
        jnp.dot(h, w32_ref[...], preferred_element_type=jnp.float32) + b32_ref[...]
    )


def _make_gather(n_chunks, out_dim):
    mesh = plsc.VectorSubcoreMesh(core_axis_name="c", subcore_axis_name="s")

    @functools.partial(
        pl.kernel,
        mesh=mesh,
        out_type=jax.ShapeDtypeStruct((NW, n_chunks, CHUNK, out_dim),
                                      jnp.float32),
        scratch_types=[
            pltpu.VMEM((n_chunks, CHUNK), jnp.int32),
            pltpu.VMEM((n_chunks, CHUNK, out_dim), jnp.float32),
            pltpu.SemaphoreType.DMA,
        ],
    )
    def gather(table_hbm, idx_hbm, out_hbm, idx_v, rows_v, sem):
        wid = lax.axis_index("s") * NC + lax.axis_index("c")
        pltpu.sync_copy(idx_hbm.at[wid], idx_v)
        copies = [
            pltpu.async_copy(table_hbm.at[idx_v.at[j]], rows_v.at[j], sem)
            for j in range(n_chunks)
        ]
        for c in copies:
            c.wait()
        pltpu.sync_copy(rows_v, out_hbm.at[wid])

    return gather


def kernel(x, emb, W2, b2, W31, b31, W32, b32):
    batch = x.shape[0]
    lat = W31.shape[1]
    out_dim = 2 * lat
    n_chunks = batch // (NW * CHUNK)

    emb_p = jnp.pad(emb, ((0, VPAD - emb.shape[0]), (0, 0)))
    table = pl.pallas_call(
        _table_body,
        out_shape=jax.ShapeDtypeStruct((VPAD, out_dim), jnp.float32),
    )(emb_p, W2, b2.reshape(1, -1), W31, b31.reshape(1, -1), W32,
      b32.reshape(1, -1))

    idx = jnp.clip(x, 0, emb.shape[0] - 1).reshape(NW, n_chunks, CHUNK)
    out = _make_gather(n_chunks, out_dim)(table, idx)
    return out.reshape(batch, out_dim)


# speedup vs baseline: 3.9826x; 3.1368x over previous
import functools

import jax
import jax.numpy as jnp
from jax import lax
from jax.experimental import pallas as pl
from jax.experimental.pallas import tpu as pltpu
from jax.experimental.pallas import tpu_sc as plsc

VOCAB = 10
VPAD = 16
NC = 2
NS = 16
NW = NC * NS
CHUNK = 128


def _table_body(emb_ref, w2_ref, b2_ref, w31_ref, b31_ref, w32_ref, b32_ref,
                out_ref):
    e = emb_ref[...]
    t = e * jax.nn.sigmoid(e)
    h = jnp.dot(t, w2_ref[...], preferred_element_type=jnp.float32) + b2_ref[...]
    h = h * jax.nn.sigmoid(h)
    lat = w31_ref.shape[1]
    out_ref[:, :lat] = (
        jnp.dot(h, w31_ref[...], preferred_element_type=jnp.float32) + b31_ref[...]
    )
    out_ref[:, lat:] = (
        jnp.dot(h, w32_ref[...], preferred_element_type=jnp.float32) + b32_ref[...]
    )


def _make_gather(n_chunks, out_dim):
    mesh = plsc.VectorSubcoreMesh(core_axis_name="c", subcore_axis_name="s")

    @functools.partial(
        pl.kernel,
        mesh=mesh,
        out_type=jax.ShapeDtypeStruct((NW, n_chunks, CHUNK, out_dim),
                                      jnp.float32),
        scratch_types=[
            pltpu.VMEM((n_chunks, CHUNK), jnp.int32),
            pltpu.VMEM((n_chunks, CHUNK, out_dim), jnp.float32),
            pltpu.VMEM((VPAD, out_dim), jnp.float32),
            pltpu.VMEM_SHARED((VPAD, out_dim), jnp.float32),
            pltpu.SemaphoreType.DMA,
        ],
    )
    def gather(table_hbm, idx_hbm, out_hbm, idx_v, rows_v, table_v, table_sh,
               sem):
        sid = lax.axis_index("s")
        wid = sid * NC + lax.axis_index("c")
        @pl.when(sid == 0)
        def _stage():
            pltpu.sync_copy(table_hbm, table_v)
            pltpu.sync_copy(table_v, table_sh)

        pltpu.sync_copy(idx_hbm.at[wid], idx_v)
        plsc.subcore_barrier()
        copies = [
            pltpu.async_copy(table_sh.at[idx_v.at[j]], rows_v.at[j], sem)
            for j in range(n_chunks)
        ]
        for c in copies:
            c.wait()
        pltpu.sync_copy(rows_v, out_hbm.at[wid])

    return gather


def kernel(x, emb, W2, b2, W31, b31, W32, b32):
    batch = x.shape[0]
    lat = W31.shape[1]
    out_dim = 2 * lat
    n_chunks = batch // (NW * CHUNK)

    emb_p = jnp.pad(emb, ((0, VPAD - emb.shape[0]), (0, 0)))
    table = pl.pallas_call(
        _table_body,
        out_shape=jax.ShapeDtypeStruct((VPAD, out_dim), jnp.float32),
    )(emb_p, W2, b2.reshape(1, -1), W31, b31.reshape(1, -1), W32,
      b32.reshape(1, -1))

    idx = jnp.clip(x, 0, emb.shape[0] - 1).reshape(NW, n_chunks, CHUNK)
    out = _make_gather(n_chunks, out_dim)(table, idx)
    return out.reshape(batch, out_dim)


# speedup vs baseline: 4.5917x; 1.1529x over previous
import functools

import jax
import jax.numpy as jnp
from jax import lax
from jax.experimental import pallas as pl
from jax.experimental.pallas import tpu as pltpu
from jax.experimental.pallas import tpu_sc as plsc

NC = 2
NS = 16
NW = NC * NS
CHUNK = 128


def _table_body(emb_ref, w2_ref, b2_ref, w31_ref, b31_ref, w32_ref, b32_ref,
                out_ref):
    e = emb_ref[...]
    t = e * jax.nn.sigmoid(e)
    h = jnp.dot(t, w2_ref[...], preferred_element_type=jnp.float32) + b2_ref[...]
    h = h * jax.nn.sigmoid(h)
    lat = w31_ref.shape[1]
    out_ref[:, :lat] = (
        jnp.dot(h, w31_ref[...], preferred_element_type=jnp.float32) + b31_ref[...]
    )
    out_ref[:, lat:] = (
        jnp.dot(h, w32_ref[...], preferred_element_type=jnp.float32) + b32_ref[...]
    )


def _make_gather(vocab, n_chunks, out_dim):
    mesh = plsc.VectorSubcoreMesh(core_axis_name="c", subcore_axis_name="s")

    @functools.partial(
        pl.kernel,
        mesh=mesh,
        out_type=jax.ShapeDtypeStruct((NW, n_chunks, CHUNK, out_dim),
                                      jnp.float32),
        scratch_types=[
            pltpu.VMEM((n_chunks, CHUNK), jnp.int32),
            pltpu.VMEM((n_chunks, CHUNK, out_dim), jnp.float32),
            pltpu.VMEM((vocab, out_dim), jnp.float32),
            pltpu.VMEM_SHARED((vocab, out_dim), jnp.float32),
            pltpu.SemaphoreType.DMA,
            pltpu.SemaphoreType.DMA,
        ],
    )
    def gather(table_hbm, idx_hbm, out_hbm, idx_v, rows_v, table_v, table_sh,
               gsem, wsem):
        sid = lax.axis_index("s")
        wid = sid * NC + lax.axis_index("c")
        @pl.when(sid == 0)
        def _stage():
            pltpu.sync_copy(table_hbm, table_v)
            pltpu.sync_copy(table_v, table_sh)

        pltpu.sync_copy(idx_hbm.at[wid], idx_v)
        plsc.subcore_barrier()
        gathers = [
            pltpu.async_copy(table_sh.at[idx_v.at[j]], rows_v.at[j], gsem)
            for j in range(n_chunks)
        ]
        writes = []
        for j in range(n_chunks):
            gathers[j].wait()
            writes.append(
                pltpu.async_copy(rows_v.at[j], out_hbm.at[wid].at[j], wsem))
        for w in writes:
            w.wait()

    return gather


def kernel(x, emb, W2, b2, W31, b31, W32, b32):
    batch = x.shape[0]
    vocab = emb.shape[0]
    lat = W31.shape[1]
    out_dim = 2 * lat
    n_chunks = batch // (NW * CHUNK)

    table = pl.pallas_call(
        _table_body,
        out_shape=jax.ShapeDtypeStruct((vocab, out_dim), jnp.float32),
    )(emb, W2, b2.reshape(1, -1), W31, b31.reshape(1, -1), W32,
      b32.reshape(1, -1))

    idx = x.reshape(NW, n_chunks, CHUNK)
    out = _make_gather(vocab, n_chunks, out_dim)(table, idx)
    return out.reshape(batch, out_dim)
